# baseline (speedup 1.0000x reference)
import functools

import jax
import jax.numpy as jnp
from jax import lax
from jax.experimental import pallas as pl
from jax.experimental.pallas import tpu as pltpu
from jax.experimental.pallas import tpu_sc as plsc

USER_LEN = 1000000
MOVIE_LEN = 100000
EMBED = 32
BATCH = 16384

RRELU_SLOPE = (1.0 / 8.0 + 1.0 / 3.0) / 2.0

_INFO = plsc.get_sparse_core_info()
_NC = _INFO.num_cores
_NS = _INFO.num_subcores
_NW = _NC * _NS
_B_PER_W = BATCH // _NW
_CHUNK = 128
_NCHUNK = _B_PER_W // _CHUNK


def _rrelu(x):
    return jnp.where(x >= 0, x, x * RRELU_SLOPE)


def _sc_kernel(seq0_hbm, seq1_hbm, w0_hbm, w1_hbm, b0_hbm, b1_hbm,
               out_hbm,
               idx0_v, idx1_v, rows0_v, rows1_v, bias0_v, bias1_v,
               out_v, sem):
    wid = lax.axis_index("s") * _NC + lax.axis_index("c")

    row0 = wid * _NCHUNK
    pltpu.sync_copy(seq0_hbm.at[pl.ds(row0, _NCHUNK)], idx0_v)
    pltpu.sync_copy(seq1_hbm.at[pl.ds(row0, _NCHUNK)], idx1_v)

    copies = []
    for j in range(_NCHUNK):
        dst = pl.ds(j * _CHUNK, _CHUNK)
        copies.append(pltpu.async_copy(
            w0_hbm.at[idx0_v.at[j]], rows0_v.at[dst], sem))
        copies.append(pltpu.async_copy(
            w1_hbm.at[idx1_v.at[j]], rows1_v.at[dst], sem))
        copies.append(pltpu.async_copy(
            b0_hbm.at[idx0_v.at[j]], bias0_v.at[dst], sem))
        copies.append(pltpu.async_copy(
            b1_hbm.at[idx1_v.at[j]], bias1_v.at[dst], sem))
    for c in copies:
        c.wait()

    lane = lax.iota(jnp.int32, 16)

    def body(g, carry):
        base = g * 16
        item = base + lane
        acc = bias0_v[pl.ds(base, 16)] + bias1_v[pl.ds(base, 16)]
        for e in range(EMBED):
            ee = jnp.full((16,), e, jnp.int32)
            g0 = plsc.load_gather(rows0_v, [item, ee])
            g1 = plsc.load_gather(rows1_v, [item, ee])
            acc = acc + _rrelu(g0) * _rrelu(g1)
        out_v[pl.ds(base, 16)] = acc
        return carry

    lax.fori_loop(0, _B_PER_W // 16, body, 0)

    pltpu.sync_copy(out_v, out_hbm.at[pl.ds(wid * _B_PER_W, _B_PER_W)])


@functools.partial(
    pl.kernel,
    out_type=jax.ShapeDtypeStruct((BATCH,), jnp.float32),
    mesh=plsc.VectorSubcoreMesh(core_axis_name="c", subcore_axis_name="s"),
    compiler_params=pltpu.CompilerParams(
        needs_layout_passes=False, use_tc_tiling_on_sc=False),
    scratch_types=[
        pltpu.VMEM((_NCHUNK, _CHUNK), jnp.int32),
        pltpu.VMEM((_NCHUNK, _CHUNK), jnp.int32),
        pltpu.VMEM((_B_PER_W, EMBED), jnp.float32),
        pltpu.VMEM((_B_PER_W, EMBED), jnp.float32),
        pltpu.VMEM((_B_PER_W,), jnp.float32),
        pltpu.VMEM((_B_PER_W,), jnp.float32),
        pltpu.VMEM((_B_PER_W,), jnp.float32),
        pltpu.SemaphoreType.DMA,
    ],
)
def _mf_forward(seq0, seq1, w0, w1, b0, b1, out, *scratch):
    _sc_kernel(seq0, seq1, w0, w1, b0, b1, out, *scratch)


def kernel(seq0, seq1, W0, W1, B0, B1):
    seq0 = seq0.astype(jnp.int32).reshape(BATCH // _CHUNK, _CHUNK)
    seq1 = seq1.astype(jnp.int32).reshape(BATCH // _CHUNK, _CHUNK)
    out = _mf_forward(seq0, seq1, W0, W1, B0.reshape(-1), B1.reshape(-1))
    return out.reshape(BATCH, 1)
